# Initial kernel scaffold; baseline (speedup 1.0000x reference)
import jax
import jax.numpy as jnp
from jax.experimental import pallas as pl


def kernel(x, W):
    raise NotImplementedError("write your pallas kernel here")


# speedup vs baseline: 4.6684x; 4.6684x over previous
import jax
import jax.numpy as jnp
from jax.experimental import pallas as pl
from jax.experimental.pallas import tpu as pltpu

_TOKENS = 16384
_D_MODEL = 2048
_EXPERTS = 64
_BT = 1024


def _router_block(x_ref, w_ref, out_ref):
    x = x_ref[...]
    w = w_ref[...]
    logits = jax.lax.dot_general(
        x, w, (((1,), (1,)), ((), ())), preferred_element_type=jnp.float32
    )
    iota = jax.lax.broadcasted_iota(jnp.int32, logits.shape, 1)
    m1 = jnp.max(logits, axis=1, keepdims=True)
    i1 = jnp.min(jnp.where(logits == m1, iota, _EXPERTS), axis=1, keepdims=True)
    is1 = iota == i1
    masked = jnp.where(is1, -jnp.inf, logits)
    m2 = jnp.max(masked, axis=1, keepdims=True)
    i2 = jnp.min(jnp.where(masked == m2, iota, _EXPERTS), axis=1, keepdims=True)
    g1 = jax.nn.sigmoid(m1 - m2)
    out_ref[...] = jnp.where(is1, g1, 0.0) + jnp.where(iota == i2, 1.0 - g1, 0.0)


def kernel(x, W):
    grid = (_TOKENS // _BT,)
    return pl.pallas_call(
        _router_block,
        grid=grid,
        in_specs=[
            pl.BlockSpec((_BT, _D_MODEL), lambda i: (i, 0)),
            pl.BlockSpec((_EXPERTS, _D_MODEL), lambda i: (0, 0)),
        ],
        out_specs=pl.BlockSpec((_BT, _EXPERTS), lambda i: (i, 0)),
        out_shape=jax.ShapeDtypeStruct((_TOKENS, _EXPERTS), jnp.float32),
        compiler_params=pltpu.CompilerParams(
            dimension_semantics=("arbitrary",),
        ),
    )(x, W)


# speedup vs baseline: 4.8994x; 1.0495x over previous
import jax
import jax.numpy as jnp
from jax.experimental import pallas as pl
from jax.experimental.pallas import tpu as pltpu

_TOKENS = 16384
_D_MODEL = 2048
_EXPERTS = 64
_BT = 2048


def _router_block(x_ref, w_ref, out_ref):
    x = x_ref[...]
    w = w_ref[...]
    logits = jax.lax.dot_general(
        x, w, (((1,), (1,)), ((), ())), preferred_element_type=jnp.float32
    )
    iota = jax.lax.broadcasted_iota(jnp.int32, logits.shape, 1)
    m1 = jnp.max(logits, axis=1, keepdims=True)
    i1 = jnp.min(jnp.where(logits == m1, iota, _EXPERTS), axis=1, keepdims=True)
    is1 = iota == i1
    masked = jnp.where(is1, -jnp.inf, logits)
    m2 = jnp.max(masked, axis=1, keepdims=True)
    i2 = jnp.min(jnp.where(masked == m2, iota, _EXPERTS), axis=1, keepdims=True)
    g1 = jax.nn.sigmoid(m1 - m2)
    out_ref[...] = jnp.where(is1, g1, 0.0) + jnp.where(iota == i2, 1.0 - g1, 0.0)


def kernel(x, W):
    grid = (_TOKENS // _BT,)
    return pl.pallas_call(
        _router_block,
        grid=grid,
        in_specs=[
            pl.BlockSpec((_BT, _D_MODEL), lambda i: (i, 0)),
            pl.BlockSpec((_EXPERTS, _D_MODEL), lambda i: (0, 0)),
        ],
        out_specs=pl.BlockSpec((_BT, _EXPERTS), lambda i: (i, 0)),
        out_shape=jax.ShapeDtypeStruct((_TOKENS, _EXPERTS), jnp.float32),
        compiler_params=pltpu.CompilerParams(
            dimension_semantics=("arbitrary",),
        ),
    )(x, W)
